# baseline (speedup 1.0000x reference)
import functools

import jax
import jax.numpy as jnp
from jax import lax
from jax.experimental import pallas as pl
from jax.experimental.pallas import tpu as pltpu
from jax.experimental.pallas import tpu_sc as plsc

N_CODES = 8192
DIM = 256

BX = 1024
BK = 2048


def _argmin_body(x_ref, xsq_ref, w_ref, esq_ref, out_ref, accv_ref, acci_ref):
    j = pl.program_id(1)
    nk = pl.num_programs(1)

    @pl.when(j == 0)
    def _init():
        accv_ref[...] = jnp.full((1, BX), jnp.inf, jnp.float32)
        acci_ref[...] = jnp.zeros((1, BX), jnp.int32)

    s = lax.dot_general(x_ref[...], w_ref[...], (((1,), (1,)), ((), ())),
                        preferred_element_type=jnp.float32)
    d2 = (xsq_ref[...] - 2.0 * s) + esq_ref[0]
    dist = jnp.sqrt(jnp.maximum(d2, 0.0))
    m = jnp.min(dist, axis=1)
    ids = lax.broadcasted_iota(jnp.int32, (BX, BK), 1) + j * BK
    a = jnp.min(jnp.where(dist == m[:, None], ids, jnp.int32(2**30)),
                axis=1)
    acc_v = accv_ref[0, :]
    acc_i = acci_ref[0, :]
    lt = m < acc_v
    upd_i = lt | ((m == acc_v) & (a < acc_i))
    acci_ref[0, :] = jnp.where(upd_i, a, acc_i)
    accv_ref[0, :] = jnp.where(
        lt, m.astype(jnp.bfloat16).astype(jnp.float32), acc_v)

    @pl.when(j == nk - 1)
    def _emit():
        out_ref[0, 0, :] = acci_ref[0, :]


def _argmin_indices(flat_x, x_sq, weight, e_sq):
    n = flat_x.shape[0]
    nx, nk = n // BX, N_CODES // BK
    e_sq3 = e_sq.reshape(nk, 1, BK)
    out = pl.pallas_call(
        _argmin_body,
        grid=(nx, nk),
        in_specs=[
            pl.BlockSpec((BX, DIM), lambda i, j: (i, 0)),
            pl.BlockSpec((BX, 1), lambda i, j: (i, 0)),
            pl.BlockSpec((BK, DIM), lambda i, j: (j, 0)),
            pl.BlockSpec((1, 1, BK), lambda i, j: (j, 0, 0)),
        ],
        out_specs=pl.BlockSpec((1, 1, BX), lambda i, j: (i, 0, 0)),
        out_shape=jax.ShapeDtypeStruct((nx, 1, BX), jnp.int32),
        scratch_shapes=[
            pltpu.VMEM((1, BX), jnp.float32),
            pltpu.VMEM((1, BX), jnp.int32),
        ],
        compiler_params=pltpu.CompilerParams(
            dimension_semantics=("arbitrary", "arbitrary")),
    )(flat_x, x_sq, weight, e_sq3)
    return out.reshape(n)


def _sc_gather(weight, idx):
    info = plsc.get_sparse_core_info()
    nw = info.num_cores * info.num_subcores
    b = idx.shape[0]
    b_per_w = b // nw
    mesh = plsc.VectorSubcoreMesh(core_axis_name="c", subcore_axis_name="s")

    @functools.partial(
        pl.kernel, mesh=mesh,
        out_type=jax.ShapeDtypeStruct((b, DIM), jnp.float32),
        scratch_types=[
            pltpu.VMEM((b_per_w,), jnp.int32),
            pltpu.VMEM((b_per_w, DIM), jnp.float32),
            pltpu.SemaphoreType.DMA,
        ],
    )
    def gather_k(table_hbm, idx_hbm, out_hbm, idx_v, rows_v, sem):
        wid = lax.axis_index("s") * info.num_cores + lax.axis_index("c")
        base = wid * b_per_w
        pltpu.sync_copy(idx_hbm.at[pl.ds(base, b_per_w)], idx_v)
        pltpu.async_copy(table_hbm.at[idx_v], rows_v, sem).wait()
        pltpu.sync_copy(rows_v, out_hbm.at[pl.ds(base, b_per_w)])

    return gather_k(weight, idx)


def _loss_qst_body(x_ref, q_ref, loss_ref, qst_ref):
    x = x_ref[...]
    d = q_ref[...] - x
    qst_ref[...] = x + d
    dd = d * d
    loss_ref[...] = dd + dd * 0.25


def _loss_qst(flat_x, quantized):
    n = flat_x.shape[0]
    nx = n // BX
    return pl.pallas_call(
        _loss_qst_body,
        grid=(nx,),
        in_specs=[
            pl.BlockSpec((BX, DIM), lambda i: (i, 0)),
            pl.BlockSpec((BX, DIM), lambda i: (i, 0)),
        ],
        out_specs=[
            pl.BlockSpec((BX, DIM), lambda i: (i, 0)),
            pl.BlockSpec((BX, DIM), lambda i: (i, 0)),
        ],
        out_shape=[
            jax.ShapeDtypeStruct((n, DIM), jnp.float32),
            jax.ShapeDtypeStruct((n, DIM), jnp.float32),
        ],
    )(flat_x, quantized)


def kernel(inputs, weight):
    flat_x = inputs.reshape(-1, DIM)
    x_sq = jnp.sum(flat_x ** 2, axis=1, keepdims=True)
    e_sq = jnp.sum(weight ** 2, axis=1)[None, :]
    idx = _argmin_indices(flat_x, x_sq, weight, e_sq)
    quantized = _sc_gather(weight, idx)
    loss, qst = _loss_qst(flat_x, quantized)
    return (loss.reshape(inputs.shape), qst.reshape(inputs.shape))


# speedup vs baseline: 1.7747x; 1.7747x over previous
import functools

import jax
import jax.numpy as jnp
from jax import lax
from jax.experimental import pallas as pl
from jax.experimental.pallas import tpu as pltpu
from jax.experimental.pallas import tpu_sc as plsc

N_CODES = 8192
DIM = 256

BX = 1024
BK = 2048


def _argmin_body(x_ref, xsq_ref, w2_ref, esq_ref, out_ref, accv_ref, acci_ref):
    j = pl.program_id(1)
    nk = pl.num_programs(1)

    @pl.when(j == 0)
    def _init():
        accv_ref[...] = jnp.full((1, BX), jnp.inf, jnp.float32)
        acci_ref[...] = jnp.zeros((1, BX), jnp.int32)

    s2 = lax.dot_general(w2_ref[...], x_ref[...], (((1,), (1,)), ((), ())),
                         preferred_element_type=jnp.float32)
    d2 = (xsq_ref[0] - s2) + esq_ref[...]
    m2 = jnp.min(d2, axis=0, keepdims=True)
    m2c = jnp.maximum(m2, 0.0)
    dmin = jnp.sqrt(m2c)
    gbits = lax.bitcast_convert_type(dmin * dmin, jnp.int32)
    u = m2c
    for k in range(-6, 8):
        cand = lax.bitcast_convert_type(gbits + k, jnp.float32)
        hit = jnp.sqrt(jnp.maximum(cand, 0.0)) == dmin
        u = jnp.where(hit, jnp.maximum(u, cand), u)
    ids = lax.broadcasted_iota(jnp.int32, (BK, BX), 0)
    a = jnp.min(jnp.where(d2 <= u, ids, jnp.int32(2**30)), axis=0,
                keepdims=True) + j * BK
    acc_v = accv_ref[...]
    acc_i = acci_ref[...]
    lt = dmin < acc_v
    upd_i = lt | ((dmin == acc_v) & (a < acc_i))
    acci_ref[...] = jnp.where(upd_i, a, acc_i)
    accv_ref[...] = jnp.where(
        lt, dmin.astype(jnp.bfloat16).astype(jnp.float32), acc_v)

    @pl.when(j == nk - 1)
    def _emit():
        out_ref[0, 0, :] = acci_ref[0, :]


def _argmin_indices(flat_x, x_sq, weight2, e_sq):
    n = flat_x.shape[0]
    nx, nk = n // BX, N_CODES // BK
    xsq3 = x_sq.reshape(nx, 1, BX)
    esq_col = e_sq.reshape(N_CODES, 1)
    out = pl.pallas_call(
        _argmin_body,
        grid=(nx, nk),
        in_specs=[
            pl.BlockSpec((BX, DIM), lambda i, j: (i, 0)),
            pl.BlockSpec((1, 1, BX), lambda i, j: (i, 0, 0)),
            pl.BlockSpec((BK, DIM), lambda i, j: (j, 0)),
            pl.BlockSpec((BK, 1), lambda i, j: (j, 0)),
        ],
        out_specs=pl.BlockSpec((1, 1, BX), lambda i, j: (i, 0, 0)),
        out_shape=jax.ShapeDtypeStruct((nx, 1, BX), jnp.int32),
        scratch_shapes=[
            pltpu.VMEM((1, BX), jnp.float32),
            pltpu.VMEM((1, BX), jnp.int32),
        ],
        compiler_params=pltpu.CompilerParams(
            dimension_semantics=("arbitrary", "arbitrary")),
    )(flat_x, xsq3, weight2, esq_col)
    return out.reshape(n)


def _sc_gather(weight, idx):
    info = plsc.get_sparse_core_info()
    nw = info.num_cores * info.num_subcores
    b = idx.shape[0]
    b_per_w = b // nw
    mesh = plsc.VectorSubcoreMesh(core_axis_name="c", subcore_axis_name="s")

    @functools.partial(
        pl.kernel, mesh=mesh,
        out_type=jax.ShapeDtypeStruct((b, DIM), jnp.float32),
        scratch_types=[
            pltpu.VMEM((b_per_w,), jnp.int32),
            pltpu.VMEM((b_per_w, DIM), jnp.float32),
            pltpu.SemaphoreType.DMA,
        ],
    )
    def gather_k(table_hbm, idx_hbm, out_hbm, idx_v, rows_v, sem):
        wid = lax.axis_index("s") * info.num_cores + lax.axis_index("c")
        base = wid * b_per_w
        pltpu.sync_copy(idx_hbm.at[pl.ds(base, b_per_w)], idx_v)
        pltpu.async_copy(table_hbm.at[idx_v], rows_v, sem).wait()
        pltpu.sync_copy(rows_v, out_hbm.at[pl.ds(base, b_per_w)])

    return gather_k(weight, idx)


def _loss_qst_body(x_ref, q_ref, loss_ref, qst_ref):
    x = x_ref[...]
    d = q_ref[...] - x
    qst_ref[...] = x + d
    dd = d * d
    loss_ref[...] = dd + dd * 0.25


def _loss_qst(flat_x, quantized):
    n = flat_x.shape[0]
    nx = n // BX
    return pl.pallas_call(
        _loss_qst_body,
        grid=(nx,),
        in_specs=[
            pl.BlockSpec((BX, DIM), lambda i: (i, 0)),
            pl.BlockSpec((BX, DIM), lambda i: (i, 0)),
        ],
        out_specs=[
            pl.BlockSpec((BX, DIM), lambda i: (i, 0)),
            pl.BlockSpec((BX, DIM), lambda i: (i, 0)),
        ],
        out_shape=[
            jax.ShapeDtypeStruct((n, DIM), jnp.float32),
            jax.ShapeDtypeStruct((n, DIM), jnp.float32),
        ],
    )(flat_x, quantized)


def kernel(inputs, weight):
    flat_x = inputs.reshape(-1, DIM)
    x_sq = jnp.sum(flat_x ** 2, axis=1, keepdims=True)
    e_sq = jnp.sum(weight ** 2, axis=1)[None, :]
    idx = _argmin_indices(flat_x, x_sq, weight + weight, e_sq)
    quantized = _sc_gather(weight, idx)
    loss, qst = _loss_qst(flat_x, quantized)
    return (loss.reshape(inputs.shape), qst.reshape(inputs.shape))
